# baseline (speedup 1.0000x reference)
import functools

import jax
import jax.numpy as jnp
from jax import lax
from jax.experimental import pallas as pl
from jax.experimental.pallas import tpu as pltpu
from jax.experimental.pallas import tpu_sc as plsc

BATCH = 16384
SEQ = 50
D_AGENT = 64
D_TEMP = 16
D_OUT = D_AGENT + D_TEMP

NUM_WORKERS = 32
BBLK = 128
NBT = BATCH // BBLK
BT_PER_W = NBT // NUM_WORKERS
B_PER_W = BT_PER_W * BBLK
NBUF = 2
SGRP = SEQ // NBUF


def _body(ids_t, wa, wt, barr_h, dt_h, out,
          ids_v, arows_v, trp_v, twrows_v, trp16_v, barr, dt, gsem, wsem):
    wid = lax.axis_index("s") * 2 + lax.axis_index("c")
    col0 = wid * B_PER_W

    pltpu.sync_copy(ids_t.at[:, pl.ds(col0, B_PER_W)], ids_v)
    pltpu.sync_copy(barr_h, barr)
    pltpu.sync_copy(dt_h, dt)

    def transpose_block(src, dst, nd):
        def per_d(d, carry):
            d16 = dt[d, :]
            for g in range(BBLK // 16):
                b16 = barr[g, :]
                vals = plsc.load_gather(src, [b16, d16])
                dst[d // 8, lax.rem(d, 8), pl.ds(g * 16, 16)] = vals
            return carry

        lax.fori_loop(0, nd, per_d, 0)

    for btl in range(BT_PER_W):
        bt = wid * BT_PER_W + btl

        pltpu.async_copy(
            wt.at[ids_v.at[0, pl.ds(btl * BBLK, BBLK)]], twrows_v,
            gsem.at[0]).wait()
        for d in range(D_TEMP):
            d16 = dt[d, :]
            for g in range(BBLK // 16):
                b16 = barr[g, :]
                vals = plsc.load_gather(twrows_v, [b16, d16])
                trp16_v[d // 8, d % 8, pl.ds(g * 16, 16)] = vals

        def copyA(slot, s):
            return pltpu.make_async_copy(
                wa.at[ids_v.at[s, pl.ds(btl * BBLK, BBLK)]],
                arows_v.at[slot], gsem.at[slot])

        for sl in range(NBUF):
            copyA(sl, sl).start()

        def sgroup(g, carry):
            for sl in range(NBUF):
                s = g * NBUF + sl
                copyA(sl, s).wait()

                @pl.when(g > 0)
                def _wait_prev():
                    pltpu.make_async_copy(
                        trp_v.at[sl], out.at[0, pl.ds(0, 8), bt],
                        wsem.at[sl]).wait()
                    pltpu.make_async_copy(
                        trp16_v, out.at[0, pl.ds(8, 2), bt],
                        wsem.at[sl]).wait()

                transpose_block(arows_v.at[sl], trp_v.at[sl], D_AGENT)
                pltpu.async_copy(
                    trp_v.at[sl],
                    out.at[s, pl.ds(0, 8), bt], wsem.at[sl])
                pltpu.async_copy(
                    trp16_v,
                    out.at[s, pl.ds(8, 2), bt], wsem.at[sl])
                sn = jnp.minimum(s + NBUF, SEQ - 1)
                copyA(sl, sn).start()
            return carry

        lax.fori_loop(0, SGRP, sgroup, 0)

        for sl in range(NBUF):
            copyA(sl, SEQ - 1).wait()
            pltpu.make_async_copy(
                trp_v.at[sl], out.at[SEQ - 1, pl.ds(0, 8), bt],
                wsem.at[sl]).wait()
            pltpu.make_async_copy(
                trp16_v, out.at[SEQ - 1, pl.ds(8, 2), bt],
                wsem.at[sl]).wait()


@functools.partial(jax.jit, static_argnums=())
def kernel(agent_ids, W_agent, W_temp):
    ids_t = agent_ids.T.astype(jnp.int32)
    barr = jnp.arange(BBLK, dtype=jnp.int32).reshape(BBLK // 16, 16)
    dt = jnp.broadcast_to(
        jnp.arange(D_AGENT, dtype=jnp.int32)[:, None], (D_AGENT, 16))

    run = pl.kernel(
        _body,
        out_type=jax.ShapeDtypeStruct(
            (SEQ, D_OUT // 8, NBT, 8, BBLK), jnp.float32),
        mesh=plsc.VectorSubcoreMesh(core_axis_name="c", subcore_axis_name="s"),
        scratch_types=[
            pltpu.VMEM((SEQ, B_PER_W), jnp.int32),
            pltpu.VMEM((NBUF, BBLK, D_AGENT), jnp.float32),
            pltpu.VMEM((NBUF, D_AGENT // 8, 8, BBLK), jnp.float32),
            pltpu.VMEM((BBLK, D_TEMP), jnp.float32),
            pltpu.VMEM((D_TEMP // 8, 8, BBLK), jnp.float32),
            pltpu.VMEM((BBLK // 16, 16), jnp.int32),
            pltpu.VMEM((D_AGENT, 16), jnp.int32),
            pltpu.SemaphoreType.DMA((NBUF,)),
            pltpu.SemaphoreType.DMA((NBUF,)),
        ],
        compiler_params=pltpu.CompilerParams(
            use_tc_tiling_on_sc=False, needs_layout_passes=False),
    )
    out5 = run(ids_t, W_agent, W_temp, barr, dt)
    return out5.transpose(2, 4, 0, 1, 3).reshape(BATCH, SEQ, D_OUT)


# speedup vs baseline: 2.6188x; 2.6188x over previous
import functools

import jax
import jax.numpy as jnp
from jax import lax
from jax.experimental import pallas as pl
from jax.experimental.pallas import tpu as pltpu
from jax.experimental.pallas import tpu_sc as plsc

BATCH = 16384
SEQ = 50
D_AGENT = 64
D_TEMP = 16
D_OUT = D_AGENT + D_TEMP

NUM_WORKERS = 32
BBLK = 128
NBT = BATCH // BBLK
BT_PER_W = NBT // NUM_WORKERS
B_PER_W = BT_PER_W * BBLK
NBUF = 2
SGRP = SEQ // NBUF


def _body(ids_t, wa, wt, barr_h, dt_h, out,
          ids_v, arows_v, trp_v, twrows_v, trp16_v, barr, dt, gsem, wsem):
    wid = lax.axis_index("s") * 2 + lax.axis_index("c")
    col0 = wid * B_PER_W

    pltpu.sync_copy(ids_t.at[:, pl.ds(col0, B_PER_W)], ids_v)
    pltpu.sync_copy(barr_h, barr)
    pltpu.sync_copy(dt_h, dt)

    b16s = [barr[g, :] for g in range(BBLK // 16)]

    def transpose_block(src, dst, nd):
        @functools.partial(plsc.parallel_loop, 0, nd, unroll=8)
        def _per_d(d):
            d16 = dt[d, :]
            for g in range(BBLK // 16):
                vals = plsc.load_gather(src, [b16s[g], d16])
                dst[d // 8, lax.rem(d, 8), pl.ds(g * 16, 16)] = vals

    for btl in range(BT_PER_W):
        bt = wid * BT_PER_W + btl

        pltpu.async_copy(
            wt.at[ids_v.at[0, pl.ds(btl * BBLK, BBLK)]], twrows_v,
            gsem.at[0]).wait()
        for d in range(D_TEMP):
            d16 = dt[d, :]
            for g in range(BBLK // 16):
                vals = plsc.load_gather(twrows_v, [b16s[g], d16])
                trp16_v[d // 8, d % 8, pl.ds(g * 16, 16)] = vals

        def copyA(slot, s):
            return pltpu.make_async_copy(
                wa.at[ids_v.at[s, pl.ds(btl * BBLK, BBLK)]],
                arows_v.at[slot], gsem.at[slot])

        for sl in range(NBUF):
            copyA(sl, sl).start()

        def sgroup(g, carry):
            for sl in range(NBUF):
                s = g * NBUF + sl
                copyA(sl, s).wait()

                @pl.when(g > 0)
                def _wait_prev():
                    pltpu.make_async_copy(
                        trp_v.at[sl], out.at[0, pl.ds(0, 8), bt],
                        wsem.at[sl]).wait()
                    pltpu.make_async_copy(
                        trp16_v, out.at[0, pl.ds(8, 2), bt],
                        wsem.at[sl]).wait()

                transpose_block(arows_v.at[sl], trp_v.at[sl], D_AGENT)
                pltpu.async_copy(
                    trp_v.at[sl],
                    out.at[s, pl.ds(0, 8), bt], wsem.at[sl])
                pltpu.async_copy(
                    trp16_v,
                    out.at[s, pl.ds(8, 2), bt], wsem.at[sl])
                sn = jnp.minimum(s + NBUF, SEQ - 1)
                copyA(sl, sn).start()
            return carry

        lax.fori_loop(0, SGRP, sgroup, 0)

        for sl in range(NBUF):
            copyA(sl, SEQ - 1).wait()
            pltpu.make_async_copy(
                trp_v.at[sl], out.at[SEQ - 1, pl.ds(0, 8), bt],
                wsem.at[sl]).wait()
            pltpu.make_async_copy(
                trp16_v, out.at[SEQ - 1, pl.ds(8, 2), bt],
                wsem.at[sl]).wait()


@functools.partial(jax.jit, static_argnums=())
def kernel(agent_ids, W_agent, W_temp):
    ids_t = agent_ids.T.astype(jnp.int32)
    barr = jnp.arange(BBLK, dtype=jnp.int32).reshape(BBLK // 16, 16)
    dt = jnp.broadcast_to(
        jnp.arange(D_AGENT, dtype=jnp.int32)[:, None], (D_AGENT, 16))

    run = pl.kernel(
        _body,
        out_type=jax.ShapeDtypeStruct(
            (SEQ, D_OUT // 8, NBT, 8, BBLK), jnp.float32),
        mesh=plsc.VectorSubcoreMesh(core_axis_name="c", subcore_axis_name="s"),
        scratch_types=[
            pltpu.VMEM((SEQ, B_PER_W), jnp.int32),
            pltpu.VMEM((NBUF, BBLK, D_AGENT), jnp.float32),
            pltpu.VMEM((NBUF, D_AGENT // 8, 8, BBLK), jnp.float32),
            pltpu.VMEM((BBLK, D_TEMP), jnp.float32),
            pltpu.VMEM((D_TEMP // 8, 8, BBLK), jnp.float32),
            pltpu.VMEM((BBLK // 16, 16), jnp.int32),
            pltpu.VMEM((D_AGENT, 16), jnp.int32),
            pltpu.SemaphoreType.DMA((NBUF,)),
            pltpu.SemaphoreType.DMA((NBUF,)),
        ],
        compiler_params=pltpu.CompilerParams(
            use_tc_tiling_on_sc=False, needs_layout_passes=False),
    )
    out5 = run(ids_t, W_agent, W_temp, barr, dt)
    return out5.transpose(2, 4, 0, 1, 3).reshape(BATCH, SEQ, D_OUT)
